# baseline (speedup 1.0000x reference)
import jax
import jax.numpy as jnp
from jax import lax
from jax.experimental import pallas as pl
from jax.experimental.pallas import tpu as pltpu
from jax.experimental.pallas import tpu_sc as plsc

BATCH = 4096
HIST = 200
WORD_DIM = 32
B = BATCH * HIST
NW = 32
B_CHUNK = 1024
UNITS = (BATCH // B_CHUNK) * HIST
UNITS_PER_W = UNITS // NW
QPH = BATCH // B_CHUNK


def _emb_body(idx_hbm, table_hbm, out_hbm, idx_v, rows_v, tr_v, sem):
    wid = lax.axis_index("s") * 2 + lax.axis_index("c")
    lanes = lax.iota(jnp.int32, 16)

    def step(u, _):
        unit = wid * UNITS_PER_W + u
        h = unit // QPH
        b0 = (unit % QPH) * B_CHUNK
        off = pl.multiple_of(h * BATCH + b0, B_CHUNK)
        pltpu.sync_copy(idx_hbm.at[pl.ds(off, B_CHUNK)], idx_v)
        pltpu.async_copy(table_hbm.at[idx_v], rows_v, sem).wait()

        def tr_step(b16, _):
            row_idx = b16 * 16 + lanes
            for c in range(WORD_DIM):
                col_idx = jnp.full((16,), c, jnp.int32)
                vals = plsc.load_gather(rows_v, [row_idx, col_idx])
                tr_v[c, pl.ds(pl.multiple_of(b16 * 16, 16), 16)] = vals
            return 0

        lax.fori_loop(0, B_CHUNK // 16, tr_step, 0)
        pltpu.sync_copy(tr_v, out_hbm.at[h, :, pl.ds(b0, B_CHUNK)])
        return 0

    lax.fori_loop(0, UNITS_PER_W, step, 0)


@jax.jit
def _emb(idx, word_em):
    return pl.kernel(
        _emb_body,
        out_type=jax.ShapeDtypeStruct((HIST, WORD_DIM, BATCH), jnp.float32),
        mesh=plsc.VectorSubcoreMesh(core_axis_name="c", subcore_axis_name="s"),
        scratch_types=[
            pltpu.VMEM((B_CHUNK,), jnp.int32),
            pltpu.VMEM((B_CHUNK, WORD_DIM), jnp.float32),
            pltpu.VMEM((WORD_DIM, B_CHUNK), jnp.float32),
            pltpu.SemaphoreType.DMA,
        ],
        compiler_params=pltpu.CompilerParams(
            use_tc_tiling_on_sc=False, needs_layout_passes=False
        ),
    )(idx, word_em)


def kernel(review, word_em):
    idx = review.T.reshape(B).astype(jnp.int32)
    out = _emb(idx, word_em)
    return jnp.transpose(out, (2, 0, 1))


# speedup vs baseline: 1.1837x; 1.1837x over previous
import jax
import jax.numpy as jnp
from jax import lax
from jax.experimental import pallas as pl
from jax.experimental.pallas import tpu as pltpu
from jax.experimental.pallas import tpu_sc as plsc

BATCH = 4096
HIST = 200
WORD_DIM = 32
B = BATCH * HIST
NW = 32
B_CHUNK = 1024
UNITS = (BATCH // B_CHUNK) * HIST
UNITS_PER_W = UNITS // NW
QPH = BATCH // B_CHUNK


def _emb_body(idx_hbm, table_hbm, out_hbm, idx_v, rows_v, tr_v, sem):
    wid = lax.axis_index("s") * 2 + lax.axis_index("c")
    lanes = lax.iota(jnp.int32, 16)

    def step(u, _):
        unit = wid * UNITS_PER_W + u
        h = unit // QPH
        b0 = (unit % QPH) * B_CHUNK
        off = pl.multiple_of(h * BATCH + b0, B_CHUNK)
        pltpu.sync_copy(idx_hbm.at[pl.ds(off, B_CHUNK)], idx_v)
        pltpu.async_copy(table_hbm.at[idx_v], rows_v, sem).wait()

        @plsc.parallel_loop(0, B_CHUNK // 16, unroll=4)
        def _(b16):
            row_idx = b16 * 16 + lanes
            for c in range(WORD_DIM):
                col_idx = jnp.full((16,), c, jnp.int32)
                vals = plsc.load_gather(rows_v, [row_idx, col_idx])
                tr_v[c, pl.ds(pl.multiple_of(b16 * 16, 16), 16)] = vals
        pltpu.sync_copy(tr_v, out_hbm.at[h, :, pl.ds(b0, B_CHUNK)])
        return 0

    lax.fori_loop(0, UNITS_PER_W, step, 0)


@jax.jit
def _emb(idx, word_em):
    return pl.kernel(
        _emb_body,
        out_type=jax.ShapeDtypeStruct((HIST, WORD_DIM, BATCH), jnp.float32),
        mesh=plsc.VectorSubcoreMesh(core_axis_name="c", subcore_axis_name="s"),
        scratch_types=[
            pltpu.VMEM((B_CHUNK,), jnp.int32),
            pltpu.VMEM((B_CHUNK, WORD_DIM), jnp.float32),
            pltpu.VMEM((WORD_DIM, B_CHUNK), jnp.float32),
            pltpu.SemaphoreType.DMA,
        ],
        compiler_params=pltpu.CompilerParams(
            use_tc_tiling_on_sc=False, needs_layout_passes=False
        ),
    )(idx, word_em)


def kernel(review, word_em):
    idx = review.T.reshape(B).astype(jnp.int32)
    out = _emb(idx, word_em)
    return jnp.transpose(out, (2, 0, 1))


# speedup vs baseline: 1.6906x; 1.4282x over previous
import jax
import jax.numpy as jnp
from jax import lax
from jax.experimental import pallas as pl
from jax.experimental.pallas import tpu as pltpu
from jax.experimental.pallas import tpu_sc as plsc

BATCH = 4096
HIST = 200
WORD_DIM = 32
B = BATCH * HIST
NW = 32
B_CHUNK = 1024
UNITS = (BATCH // B_CHUNK) * HIST
UNITS_PER_W = UNITS // NW
QPH = BATCH // B_CHUNK
TR_STRIDE = B_CHUNK + 1


def _emb_body(idx_hbm, table_hbm, out_hbm, idx_v, rows_v, tr_v, sem):
    wid = lax.axis_index("s") * 2 + lax.axis_index("c")
    lanes = lax.iota(jnp.int32, 16)
    c_lo = lanes
    c_hi = lanes + 16

    def step(u, _):
        unit = wid * UNITS_PER_W + u
        h = unit // QPH
        b0 = (unit % QPH) * B_CHUNK
        off = pl.multiple_of(h * BATCH + b0, B_CHUNK)
        pltpu.sync_copy(idx_hbm.at[pl.ds(off, B_CHUNK)], idx_v)
        pltpu.async_copy(table_hbm.at[idx_v], rows_v, sem).wait()

        @plsc.parallel_loop(0, B_CHUNK, unroll=8)
        def _(i):
            b_idx = jnp.full((16,), 0, jnp.int32) + i
            v0 = rows_v[i, pl.ds(0, 16)]
            v1 = rows_v[i, pl.ds(16, 16)]
            plsc.store_scatter(tr_v, [c_lo, b_idx], v0)
            plsc.store_scatter(tr_v, [c_hi, b_idx], v1)

        for c in range(WORD_DIM):
            pltpu.sync_copy(
                tr_v.at[c, pl.ds(0, B_CHUNK)],
                out_hbm.at[h, c, pl.ds(b0, B_CHUNK)],
            )
        return 0

    lax.fori_loop(0, UNITS_PER_W, step, 0)


@jax.jit
def _emb(idx, word_em):
    return pl.kernel(
        _emb_body,
        out_type=jax.ShapeDtypeStruct((HIST, WORD_DIM, BATCH), jnp.float32),
        mesh=plsc.VectorSubcoreMesh(core_axis_name="c", subcore_axis_name="s"),
        scratch_types=[
            pltpu.VMEM((B_CHUNK,), jnp.int32),
            pltpu.VMEM((B_CHUNK, WORD_DIM), jnp.float32),
            pltpu.VMEM((WORD_DIM, TR_STRIDE), jnp.float32),
            pltpu.SemaphoreType.DMA,
        ],
        compiler_params=pltpu.CompilerParams(
            use_tc_tiling_on_sc=False, needs_layout_passes=False
        ),
    )(idx, word_em)


def kernel(review, word_em):
    idx = review.T.reshape(B).astype(jnp.int32)
    out = _emb(idx, word_em)
    return jnp.transpose(out, (2, 0, 1))
